# Initial kernel scaffold; baseline (speedup 1.0000x reference)
import jax
import jax.numpy as jnp
from jax.experimental import pallas as pl


def kernel(x, W_enc_parent, b_enc_parent, W_enc_leaf, b_enc_leaf, W_dec_leaf, b_dec):
    raise NotImplementedError("write your pallas kernel here")


# speedup vs baseline: 3.6698x; 3.6698x over previous
import functools

import jax
import jax.numpy as jnp
from jax.experimental import pallas as pl
from jax.experimental.pallas import tpu as pltpu

_TOPK = 32
_HI = jax.lax.Precision.DEFAULT


def _parent_body(x_ref, w_ref, be_ref, bd_ref, out_ref):
    xc = x_ref[...] - bd_ref[...]
    pre = jax.lax.dot_general(xc, w_ref[...], (((1,), (0,)), ((), ())),
                              precision=_HI, preferred_element_type=jnp.float32)
    pre = jnp.maximum(pre + be_ref[...], 0.0)
    rowmax = jnp.max(pre, axis=1, keepdims=True)
    ids = jax.lax.broadcasted_iota(jnp.int32, pre.shape, 1)
    cand = jnp.where(pre == rowmax, ids, jnp.int32(pre.shape[1]))
    amin = jnp.min(cand, axis=1, keepdims=True)
    out_ref[...] = jnp.where(ids == amin, rowmax, 0.0)


def _leaf_body(x_ref, w_ref, be_ref, bd_ref, out_ref, *, nj, cb, kk):
    j = pl.program_id(1)
    xc = x_ref[...] - bd_ref[...]
    pre = jax.lax.dot_general(xc, w_ref[...], (((1,), (0,)), ((), ())),
                              precision=_HI, preferred_element_type=jnp.float32)
    pre = jnp.maximum(pre + be_ref[...], 0.0)
    out_ref[:, pl.ds(j * cb, cb)] = pre

    @pl.when(j == nj - 1)
    def _():
        full = out_ref[...]
        xi = jax.lax.bitcast_convert_type(full, jnp.int32)
        hi0 = jnp.max(xi, axis=1, keepdims=True) + 1
        lo0 = jnp.zeros_like(hi0)

        def body(_, carry):
            lo, hi = carry
            mid = lo + (hi - lo) // 2
            cnt = jnp.sum((xi >= mid).astype(jnp.int32), axis=1, keepdims=True)
            ge = cnt >= kk
            return jnp.where(ge, mid, lo), jnp.where(ge, hi, mid)

        lo, _ = jax.lax.fori_loop(0, 31, body, (lo0, hi0))
        out_ref[...] = jnp.where(xi >= lo, full, 0.0)


def _decode_body(z_ref, w_ref, bd_ref, out_ref, acc_ref, *, nk):
    k = pl.program_id(1)

    @pl.when(k == 0)
    def _():
        acc_ref[...] = jnp.zeros_like(acc_ref)

    acc_ref[...] += jax.lax.dot_general(
        z_ref[...], w_ref[...], (((1,), (0,)), ((), ())),
        precision=_HI, preferred_element_type=jnp.float32)

    @pl.when(k == nk - 1)
    def _():
        out_ref[...] = acc_ref[...] + bd_ref[...]


def kernel(x, W_enc_parent, b_enc_parent, W_enc_leaf, b_enc_leaf, W_dec_leaf, b_dec):
    B, D_in = x.shape
    N_par = W_enc_parent.shape[1]
    D_leaf = W_enc_leaf.shape[1]
    be_p = b_enc_parent.reshape(1, N_par)
    be_l = b_enc_leaf.reshape(1, D_leaf)
    bd = b_dec.reshape(1, D_in)

    bbp = min(512, B)
    z_parent = pl.pallas_call(
        _parent_body,
        grid=(B // bbp,),
        in_specs=[
            pl.BlockSpec((bbp, D_in), lambda i: (i, 0)),
            pl.BlockSpec((D_in, N_par), lambda i: (0, 0)),
            pl.BlockSpec((1, N_par), lambda i: (0, 0)),
            pl.BlockSpec((1, D_in), lambda i: (0, 0)),
        ],
        out_specs=pl.BlockSpec((bbp, N_par), lambda i: (i, 0)),
        out_shape=jax.ShapeDtypeStruct((B, N_par), jnp.float32),
    )(x, W_enc_parent, be_p, bd)

    bb = min(128, B)
    cb = min(1024, D_leaf)
    nj = D_leaf // cb
    z_leaf = pl.pallas_call(
        functools.partial(_leaf_body, nj=nj, cb=cb, kk=_TOPK),
        grid=(B // bb, nj),
        in_specs=[
            pl.BlockSpec((bb, D_in), lambda i, j: (i, 0)),
            pl.BlockSpec((D_in, cb), lambda i, j: (0, j)),
            pl.BlockSpec((1, cb), lambda i, j: (0, j)),
            pl.BlockSpec((1, D_in), lambda i, j: (0, 0)),
        ],
        out_specs=pl.BlockSpec((bb, D_leaf), lambda i, j: (i, 0)),
        out_shape=jax.ShapeDtypeStruct((B, D_leaf), jnp.float32),
    )(x, W_enc_leaf, be_l, bd)

    bb2 = min(256, B)
    ck = min(2048, D_leaf)
    nk = D_leaf // ck
    x_hat = pl.pallas_call(
        functools.partial(_decode_body, nk=nk),
        grid=(B // bb2, nk),
        in_specs=[
            pl.BlockSpec((bb2, ck), lambda i, k: (i, k)),
            pl.BlockSpec((ck, D_in), lambda i, k: (k, 0)),
            pl.BlockSpec((1, D_in), lambda i, k: (0, 0)),
        ],
        out_specs=pl.BlockSpec((bb2, D_in), lambda i, k: (i, 0)),
        out_shape=jax.ShapeDtypeStruct((B, D_in), jnp.float32),
        scratch_shapes=[pltpu.VMEM((bb2, D_in), jnp.float32)],
    )(z_leaf, W_dec_leaf, bd)

    return (x_hat, z_parent, z_leaf)


# speedup vs baseline: 3.6723x; 1.0007x over previous
import functools

import jax
import jax.numpy as jnp
from jax.experimental import pallas as pl
from jax.experimental.pallas import tpu as pltpu

_TOPK = 32
_HI = jax.lax.Precision.DEFAULT


def _parent_body(x_ref, w_ref, be_ref, bd_ref, out_ref):
    xc = x_ref[...] - bd_ref[...]
    pre = jax.lax.dot_general(xc, w_ref[...], (((1,), (0,)), ((), ())),
                              precision=_HI, preferred_element_type=jnp.float32)
    pre = jnp.maximum(pre + be_ref[...], 0.0)
    rowmax = jnp.max(pre, axis=1, keepdims=True)
    ids = jax.lax.broadcasted_iota(jnp.int32, pre.shape, 1)
    cand = jnp.where(pre == rowmax, ids, jnp.int32(pre.shape[1]))
    amin = jnp.min(cand, axis=1, keepdims=True)
    out_ref[...] = jnp.where(ids == amin, rowmax, 0.0)


def _leaf_body(x_ref, w_ref, be_ref, bd_ref, out_ref, *, nj, cb, kk):
    j = pl.program_id(1)
    xc = x_ref[...] - bd_ref[...]
    pre = jax.lax.dot_general(xc, w_ref[...], (((1,), (0,)), ((), ())),
                              precision=_HI, preferred_element_type=jnp.float32)
    pre = jnp.maximum(pre + be_ref[...], 0.0)
    out_ref[:, pl.ds(j * cb, cb)] = pre

    @pl.when(j == nj - 1)
    def _():
        full = out_ref[...]
        xi = jax.lax.bitcast_convert_type(full, jnp.int32)
        hi0 = jnp.max(xi, axis=1, keepdims=True) + 1
        lo0 = jnp.zeros_like(hi0)

        def body(_, carry):
            lo, hi = carry
            mid = lo + (hi - lo) // 2
            cnt = jnp.sum((xi >= mid).astype(jnp.int32), axis=1, keepdims=True)
            ge = cnt >= kk
            return jnp.where(ge, mid, lo), jnp.where(ge, hi, mid)

        lo, _ = jax.lax.fori_loop(0, 31, body, (lo0, hi0))
        out_ref[...] = jnp.where(xi >= lo, full, 0.0)


def _decode_body(z_ref, w_ref, bd_ref, out_ref, acc_ref, *, nk):
    k = pl.program_id(1)

    @pl.when(k == 0)
    def _():
        acc_ref[...] = jnp.zeros_like(acc_ref)

    acc_ref[...] += jax.lax.dot_general(
        z_ref[...], w_ref[...], (((1,), (0,)), ((), ())),
        precision=_HI, preferred_element_type=jnp.float32)

    @pl.when(k == nk - 1)
    def _():
        out_ref[...] = acc_ref[...] + bd_ref[...]


def kernel(x, W_enc_parent, b_enc_parent, W_enc_leaf, b_enc_leaf, W_dec_leaf, b_dec):
    B, D_in = x.shape
    N_par = W_enc_parent.shape[1]
    D_leaf = W_enc_leaf.shape[1]
    be_p = b_enc_parent.reshape(1, N_par)
    be_l = b_enc_leaf.reshape(1, D_leaf)
    bd = b_dec.reshape(1, D_in)

    bbp = min(512, B)
    z_parent = pl.pallas_call(
        _parent_body,
        grid=(B // bbp,),
        in_specs=[
            pl.BlockSpec((bbp, D_in), lambda i: (i, 0)),
            pl.BlockSpec((D_in, N_par), lambda i: (0, 0)),
            pl.BlockSpec((1, N_par), lambda i: (0, 0)),
            pl.BlockSpec((1, D_in), lambda i: (0, 0)),
        ],
        out_specs=pl.BlockSpec((bbp, N_par), lambda i: (i, 0)),
        out_shape=jax.ShapeDtypeStruct((B, N_par), jnp.float32),
        compiler_params=pltpu.CompilerParams(
            dimension_semantics=("parallel",)),
    )(x, W_enc_parent, be_p, bd)

    bb = min(128, B)
    cb = min(1024, D_leaf)
    nj = D_leaf // cb
    z_leaf = pl.pallas_call(
        functools.partial(_leaf_body, nj=nj, cb=cb, kk=_TOPK),
        grid=(B // bb, nj),
        in_specs=[
            pl.BlockSpec((bb, D_in), lambda i, j: (i, 0)),
            pl.BlockSpec((D_in, cb), lambda i, j: (0, j)),
            pl.BlockSpec((1, cb), lambda i, j: (0, j)),
            pl.BlockSpec((1, D_in), lambda i, j: (0, 0)),
        ],
        out_specs=pl.BlockSpec((bb, D_leaf), lambda i, j: (i, 0)),
        out_shape=jax.ShapeDtypeStruct((B, D_leaf), jnp.float32),
        compiler_params=pltpu.CompilerParams(
            dimension_semantics=("parallel", "arbitrary")),
    )(x, W_enc_leaf, be_l, bd)

    bb2 = min(256, B)
    ck = min(2048, D_leaf)
    nk = D_leaf // ck
    x_hat = pl.pallas_call(
        functools.partial(_decode_body, nk=nk),
        grid=(B // bb2, nk),
        in_specs=[
            pl.BlockSpec((bb2, ck), lambda i, k: (i, k)),
            pl.BlockSpec((ck, D_in), lambda i, k: (k, 0)),
            pl.BlockSpec((1, D_in), lambda i, k: (0, 0)),
        ],
        out_specs=pl.BlockSpec((bb2, D_in), lambda i, k: (i, 0)),
        out_shape=jax.ShapeDtypeStruct((B, D_in), jnp.float32),
        scratch_shapes=[pltpu.VMEM((bb2, D_in), jnp.float32)],
        compiler_params=pltpu.CompilerParams(
            dimension_semantics=("parallel", "arbitrary")),
    )(z_leaf, W_dec_leaf, bd)

    return (x_hat, z_parent, z_leaf)


# speedup vs baseline: 4.0639x; 1.1066x over previous
import functools

import jax
import jax.numpy as jnp
from jax.experimental import pallas as pl
from jax.experimental.pallas import tpu as pltpu

_TOPK = 32
_HI = jax.lax.Precision.DEFAULT


def _parent_body(x_ref, w_ref, be_ref, bd_ref, out_ref):
    xc = x_ref[...] - bd_ref[...]
    pre = jax.lax.dot_general(xc, w_ref[...], (((1,), (0,)), ((), ())),
                              precision=_HI, preferred_element_type=jnp.float32)
    pre = jnp.maximum(pre + be_ref[...], 0.0)
    rowmax = jnp.max(pre, axis=1, keepdims=True)
    ids = jax.lax.broadcasted_iota(jnp.int32, pre.shape, 1)
    cand = jnp.where(pre == rowmax, ids, jnp.int32(pre.shape[1]))
    amin = jnp.min(cand, axis=1, keepdims=True)
    out_ref[...] = jnp.where(ids == amin, rowmax, 0.0)


def _leaf_body(x_ref, w_ref, be_ref, bd_ref, out_ref, cm_ref, *, nj, cb, kk):
    j = pl.program_id(1)
    xc = x_ref[...] - bd_ref[...]
    pre = jax.lax.dot_general(xc, w_ref[...], (((1,), (0,)), ((), ())),
                              precision=_HI, preferred_element_type=jnp.float32)
    pre = jnp.maximum(pre + be_ref[...], 0.0)
    out_ref[:, pl.ds(j * cb, cb)] = pre
    m8 = pre[:, 0:128]
    for s in range(1, cb // 128):
        m8 = jnp.maximum(m8, pre[:, s * 128:(s + 1) * 128])
    cm_ref[:, pl.ds(j * 128, 128)] = m8

    @pl.when(j == nj - 1)
    def _():
        full = out_ref[...]
        xi = jax.lax.bitcast_convert_type(full, jnp.int32)
        cmi = jax.lax.bitcast_convert_type(cm_ref[...], jnp.int32)
        hi0 = jnp.max(cmi, axis=1, keepdims=True) + 1
        lo0 = jnp.zeros_like(hi0)

        def body1(_, carry):
            lo, hi = carry
            mid = lo + (hi - lo) // 2
            cnt = jnp.sum((cmi >= mid).astype(jnp.int32), axis=1, keepdims=True)
            ge = cnt >= kk
            return jnp.where(ge, mid, lo), jnp.where(ge, hi, mid)

        lo1, _ = jax.lax.fori_loop(0, 16, body1, (lo0, hi0))

        pos = jnp.sum((xi >= 1).astype(jnp.int32), axis=1, keepdims=True)
        trivial = pos <= kk
        lo2 = jnp.where(trivial, 0, lo1)
        hi2 = jnp.where(trivial, 1, hi0)

        def cond2(carry):
            it, _, _, act = carry
            return jnp.logical_and(it < 31, act)

        def body2(carry):
            it, lo, hi, _ = carry
            mid = lo + (hi - lo) // 2
            cnt = jnp.sum((xi >= mid).astype(jnp.int32), axis=1, keepdims=True)
            ge = cnt >= kk
            eq = cnt == kk
            nlo = jnp.where(ge, mid, lo)
            nhi = jnp.where(eq, mid + 1, jnp.where(ge, hi, mid))
            act = jnp.any((nhi - nlo) > 1)
            return it + 1, nlo, nhi, act

        act0 = jnp.any((hi2 - lo2) > 1)
        _, lof, _, _ = jax.lax.while_loop(
            cond2, body2, (jnp.int32(0), lo2, hi2, act0))
        out_ref[...] = jnp.where(xi >= lof, full, 0.0)


def _decode_body(z_ref, w_ref, bd_ref, out_ref, acc_ref, *, nk):
    k = pl.program_id(1)

    @pl.when(k == 0)
    def _():
        acc_ref[...] = jnp.zeros_like(acc_ref)

    acc_ref[...] += jax.lax.dot_general(
        z_ref[...], w_ref[...], (((1,), (0,)), ((), ())),
        precision=_HI, preferred_element_type=jnp.float32)

    @pl.when(k == nk - 1)
    def _():
        out_ref[...] = acc_ref[...] + bd_ref[...]


def kernel(x, W_enc_parent, b_enc_parent, W_enc_leaf, b_enc_leaf, W_dec_leaf, b_dec):
    B, D_in = x.shape
    N_par = W_enc_parent.shape[1]
    D_leaf = W_enc_leaf.shape[1]
    be_p = b_enc_parent.reshape(1, N_par)
    be_l = b_enc_leaf.reshape(1, D_leaf)
    bd = b_dec.reshape(1, D_in)

    bbp = min(512, B)
    z_parent = pl.pallas_call(
        _parent_body,
        grid=(B // bbp,),
        in_specs=[
            pl.BlockSpec((bbp, D_in), lambda i: (i, 0)),
            pl.BlockSpec((D_in, N_par), lambda i: (0, 0)),
            pl.BlockSpec((1, N_par), lambda i: (0, 0)),
            pl.BlockSpec((1, D_in), lambda i: (0, 0)),
        ],
        out_specs=pl.BlockSpec((bbp, N_par), lambda i: (i, 0)),
        out_shape=jax.ShapeDtypeStruct((B, N_par), jnp.float32),
        compiler_params=pltpu.CompilerParams(
            dimension_semantics=("parallel",)),
    )(x, W_enc_parent, be_p, bd)

    bb = min(128, B)
    cb = min(1024, D_leaf)
    nj = D_leaf // cb
    z_leaf = pl.pallas_call(
        functools.partial(_leaf_body, nj=nj, cb=cb, kk=_TOPK),
        grid=(B // bb, nj),
        in_specs=[
            pl.BlockSpec((bb, D_in), lambda i, j: (i, 0)),
            pl.BlockSpec((D_in, cb), lambda i, j: (0, j)),
            pl.BlockSpec((1, cb), lambda i, j: (0, j)),
            pl.BlockSpec((1, D_in), lambda i, j: (0, 0)),
        ],
        out_specs=pl.BlockSpec((bb, D_leaf), lambda i, j: (i, 0)),
        out_shape=jax.ShapeDtypeStruct((B, D_leaf), jnp.float32),
        scratch_shapes=[pltpu.VMEM((bb, (D_leaf // cb) * 128), jnp.float32)],
        compiler_params=pltpu.CompilerParams(
            dimension_semantics=("parallel", "arbitrary")),
    )(x, W_enc_leaf, be_l, bd)

    bb2 = min(256, B)
    ck = min(2048, D_leaf)
    nk = D_leaf // ck
    x_hat = pl.pallas_call(
        functools.partial(_decode_body, nk=nk),
        grid=(B // bb2, nk),
        in_specs=[
            pl.BlockSpec((bb2, ck), lambda i, k: (i, k)),
            pl.BlockSpec((ck, D_in), lambda i, k: (k, 0)),
            pl.BlockSpec((1, D_in), lambda i, k: (0, 0)),
        ],
        out_specs=pl.BlockSpec((bb2, D_in), lambda i, k: (i, 0)),
        out_shape=jax.ShapeDtypeStruct((B, D_in), jnp.float32),
        scratch_shapes=[pltpu.VMEM((bb2, D_in), jnp.float32)],
        compiler_params=pltpu.CompilerParams(
            dimension_semantics=("parallel", "arbitrary")),
    )(z_leaf, W_dec_leaf, bd)

    return (x_hat, z_parent, z_leaf)


# speedup vs baseline: 7.1199x; 1.7520x over previous
import functools

import jax
import jax.numpy as jnp
from jax.experimental import pallas as pl
from jax.experimental.pallas import tpu as pltpu

_TOPK = 32


def _dot(a, b):
    return jax.lax.dot_general(a, b, (((1,), (0,)), ((), ())),
                               preferred_element_type=jnp.float32)


def _parent_body(x_ref, w_ref, be_ref, bd_ref, out_ref):
    xc = (x_ref[...] - bd_ref[...]).astype(jnp.bfloat16)
    pre = jnp.maximum(_dot(xc, w_ref[...]) + be_ref[...], 0.0)
    rowmax = jnp.max(pre, axis=1, keepdims=True)
    ids = jax.lax.broadcasted_iota(jnp.int32, pre.shape, 1)
    cand = jnp.where(pre == rowmax, ids, jnp.int32(pre.shape[1]))
    amin = jnp.min(cand, axis=1, keepdims=True)
    out_ref[...] = jnp.where(ids == amin, rowmax, 0.0)


def _leaf_enc_body(x_ref, w_ref, be_ref, bd_ref, pre_ref, cm_ref):
    xc = (x_ref[...] - bd_ref[...]).astype(jnp.bfloat16)
    pre = jnp.maximum(_dot(xc, w_ref[...]) + be_ref[...], 0.0)
    pre_ref[...] = pre
    cb = pre.shape[1]
    m8 = pre[:, 0:128]
    for s in range(1, cb // 128):
        m8 = jnp.maximum(m8, pre[:, s * 128:(s + 1) * 128])
    cm_ref[...] = m8


def _topk_body(pre_ref, cm_ref, out_ref, obf_ref, *, kk):
    full = pre_ref[...]
    xi = jax.lax.bitcast_convert_type(full, jnp.int32)
    cmi = jax.lax.bitcast_convert_type(cm_ref[...], jnp.int32)
    hi0 = jnp.max(cmi, axis=1, keepdims=True) + 1
    lo0 = jnp.zeros_like(hi0)

    def body1(_, carry):
        lo, hi = carry
        mid = lo + (hi - lo) // 2
        cnt = jnp.sum((cmi >= mid).astype(jnp.int32), axis=1, keepdims=True)
        ge = cnt >= kk
        return jnp.where(ge, mid, lo), jnp.where(ge, hi, mid)

    lo1, _ = jax.lax.fori_loop(0, 16, body1, (lo0, hi0))

    pos = jnp.sum((xi >= 1).astype(jnp.int32), axis=1, keepdims=True)
    trivial = pos <= kk
    lo2 = jnp.where(trivial, 0, lo1)
    hi2 = jnp.where(trivial, 1, hi0)

    def cond2(carry):
        it, _, _, act = carry
        return jnp.logical_and(it < 31, act)

    def body2(carry):
        it, lo, hi, _ = carry
        mid = lo + (hi - lo) // 2
        cnt = jnp.sum((xi >= mid).astype(jnp.int32), axis=1, keepdims=True)
        ge = cnt >= kk
        eq = cnt == kk
        nlo = jnp.where(ge, mid, lo)
        nhi = jnp.where(eq, mid + 1, jnp.where(ge, hi, mid))
        act = jnp.any((nhi - nlo) > 1)
        return it + 1, nlo, nhi, act

    act0 = jnp.any((hi2 - lo2) > 1)
    _, lof, _, _ = jax.lax.while_loop(
        cond2, body2, (jnp.int32(0), lo2, hi2, act0))
    z = jnp.where(xi >= lof, full, 0.0)
    out_ref[...] = z
    obf_ref[...] = z.astype(jnp.bfloat16)


def _decode_body(z_ref, w_ref, bd_ref, out_ref, acc_ref, *, nk):
    k = pl.program_id(1)

    @pl.when(k == 0)
    def _():
        acc_ref[...] = jnp.zeros_like(acc_ref)

    acc_ref[...] += _dot(z_ref[...], w_ref[...])

    @pl.when(k == nk - 1)
    def _():
        out_ref[...] = acc_ref[...] + bd_ref[...]


def kernel(x, W_enc_parent, b_enc_parent, W_enc_leaf, b_enc_leaf, W_dec_leaf, b_dec):
    B, D_in = x.shape
    N_par = W_enc_parent.shape[1]
    D_leaf = W_enc_leaf.shape[1]
    be_p = b_enc_parent.reshape(1, N_par)
    be_l = b_enc_leaf.reshape(1, D_leaf)
    bd = b_dec.reshape(1, D_in)
    wp = W_enc_parent.astype(jnp.bfloat16)
    wl = W_enc_leaf.astype(jnp.bfloat16)
    wd = W_dec_leaf.astype(jnp.bfloat16)

    bbp = min(512, B)
    z_parent = pl.pallas_call(
        _parent_body,
        grid=(B // bbp,),
        in_specs=[
            pl.BlockSpec((bbp, D_in), lambda i: (i, 0)),
            pl.BlockSpec((D_in, N_par), lambda i: (0, 0)),
            pl.BlockSpec((1, N_par), lambda i: (0, 0)),
            pl.BlockSpec((1, D_in), lambda i: (0, 0)),
        ],
        out_specs=pl.BlockSpec((bbp, N_par), lambda i: (i, 0)),
        out_shape=jax.ShapeDtypeStruct((B, N_par), jnp.float32),
        compiler_params=pltpu.CompilerParams(
            dimension_semantics=("parallel",)),
    )(x, wp, be_p, bd)

    bb = min(512, B)
    cb = min(1024, D_leaf)
    nj = D_leaf // cb
    ncm = nj * 128
    pre_leaf, cm = pl.pallas_call(
        _leaf_enc_body,
        grid=(B // bb, nj),
        in_specs=[
            pl.BlockSpec((bb, D_in), lambda i, j: (i, 0)),
            pl.BlockSpec((D_in, cb), lambda i, j: (0, j)),
            pl.BlockSpec((1, cb), lambda i, j: (0, j)),
            pl.BlockSpec((1, D_in), lambda i, j: (0, 0)),
        ],
        out_specs=[
            pl.BlockSpec((bb, cb), lambda i, j: (i, j)),
            pl.BlockSpec((bb, 128), lambda i, j: (i, j)),
        ],
        out_shape=[
            jax.ShapeDtypeStruct((B, D_leaf), jnp.float32),
            jax.ShapeDtypeStruct((B, ncm), jnp.float32),
        ],
        compiler_params=pltpu.CompilerParams(
            dimension_semantics=("parallel", "arbitrary")),
    )(x, wl, be_l, bd)

    bt = min(128, B)
    z_leaf, z_bf = pl.pallas_call(
        functools.partial(_topk_body, kk=_TOPK),
        grid=(B // bt,),
        in_specs=[
            pl.BlockSpec((bt, D_leaf), lambda i: (i, 0)),
            pl.BlockSpec((bt, ncm), lambda i: (i, 0)),
        ],
        out_specs=[
            pl.BlockSpec((bt, D_leaf), lambda i: (i, 0)),
            pl.BlockSpec((bt, D_leaf), lambda i: (i, 0)),
        ],
        out_shape=[
            jax.ShapeDtypeStruct((B, D_leaf), jnp.float32),
            jax.ShapeDtypeStruct((B, D_leaf), jnp.bfloat16),
        ],
        compiler_params=pltpu.CompilerParams(
            dimension_semantics=("parallel",)),
    )(pre_leaf, cm)

    bb2 = min(512, B)
    ck = min(2048, D_leaf)
    nk = D_leaf // ck
    x_hat = pl.pallas_call(
        functools.partial(_decode_body, nk=nk),
        grid=(B // bb2, nk),
        in_specs=[
            pl.BlockSpec((bb2, ck), lambda i, k: (i, k)),
            pl.BlockSpec((ck, D_in), lambda i, k: (k, 0)),
            pl.BlockSpec((1, D_in), lambda i, k: (0, 0)),
        ],
        out_specs=pl.BlockSpec((bb2, D_in), lambda i, k: (i, 0)),
        out_shape=jax.ShapeDtypeStruct((B, D_in), jnp.float32),
        scratch_shapes=[pltpu.VMEM((bb2, D_in), jnp.float32)],
        compiler_params=pltpu.CompilerParams(
            dimension_semantics=("parallel", "arbitrary")),
    )(z_bf, wd, bd)

    return (x_hat, z_parent, z_leaf)


# speedup vs baseline: 7.4452x; 1.0457x over previous
import functools

import jax
import jax.numpy as jnp
from jax.experimental import pallas as pl
from jax.experimental.pallas import tpu as pltpu

_TOPK = 32


def _dot(a, b):
    return jax.lax.dot_general(a, b, (((1,), (0,)), ((), ())),
                               preferred_element_type=jnp.float32)


def _parent_body(x_ref, w_ref, be_ref, bd_ref, out_ref):
    xc = (x_ref[...] - bd_ref[...]).astype(jnp.bfloat16)
    pre = jnp.maximum(_dot(xc, w_ref[...]) + be_ref[...], 0.0)
    rowmax = jnp.max(pre, axis=1, keepdims=True)
    ids = jax.lax.broadcasted_iota(jnp.int32, pre.shape, 1)
    cand = jnp.where(pre == rowmax, ids, jnp.int32(pre.shape[1]))
    amin = jnp.min(cand, axis=1, keepdims=True)
    out_ref[...] = jnp.where(ids == amin, rowmax, 0.0)


def _leaf_enc_body(x_ref, w_ref, be_ref, bd_ref, pre_ref, cm_ref):
    xc = (x_ref[...] - bd_ref[...]).astype(jnp.bfloat16)
    pre = jnp.maximum(_dot(xc, w_ref[...]) + be_ref[...], 0.0)
    pre_ref[...] = pre
    cb = pre.shape[1]
    m8 = pre[:, 0:128]
    for s in range(1, cb // 128):
        m8 = jnp.maximum(m8, pre[:, s * 128:(s + 1) * 128])
    cm_ref[...] = m8


def _topk_body(pre_ref, cm_ref, out_ref, obf_ref, *, kk):
    full = pre_ref[...]
    xi = jax.lax.bitcast_convert_type(full, jnp.int32)
    cmi = jax.lax.bitcast_convert_type(cm_ref[...], jnp.int32)
    hi0 = jnp.max(cmi, axis=1, keepdims=True) + 1
    lo0 = jnp.zeros_like(hi0)

    def body1(_, carry):
        lo, hi = carry
        mid = lo + (hi - lo) // 2
        cnt = jnp.sum((cmi >= mid).astype(jnp.int32), axis=1, keepdims=True)
        ge = cnt >= kk
        return jnp.where(ge, mid, lo), jnp.where(ge, hi, mid)

    lo1, _ = jax.lax.fori_loop(0, 16, body1, (lo0, hi0))

    pos = jnp.sum((xi >= 1).astype(jnp.int32), axis=1, keepdims=True)
    trivial = pos <= kk
    lo2 = jnp.where(trivial, 0, lo1)
    hi2 = jnp.where(trivial, 1, hi0)

    def cond2(carry):
        it, _, _, act = carry
        return jnp.logical_and(it < 31, act)

    def body2(carry):
        it, lo, hi, _ = carry
        mid = lo + (hi - lo) // 2
        cnt = jnp.sum((xi >= mid).astype(jnp.int32), axis=1, keepdims=True)
        ge = cnt >= kk
        eq = cnt == kk
        nlo = jnp.where(ge, mid, lo)
        nhi = jnp.where(eq, mid + 1, jnp.where(ge, hi, mid))
        act = jnp.any((nhi - nlo) > 1)
        return it + 1, nlo, nhi, act

    act0 = jnp.any((hi2 - lo2) > 1)
    _, lof, _, _ = jax.lax.while_loop(
        cond2, body2, (jnp.int32(0), lo2, hi2, act0))
    z = jnp.where(xi >= lof, full, 0.0)
    out_ref[...] = z
    obf_ref[...] = z.astype(jnp.bfloat16)


def _decode_body(z_ref, w_ref, bd_ref, out_ref, acc_ref, *, nk):
    k = pl.program_id(1)

    @pl.when(k == 0)
    def _():
        acc_ref[...] = jnp.zeros_like(acc_ref)

    acc_ref[...] += _dot(z_ref[...], w_ref[...])

    @pl.when(k == nk - 1)
    def _():
        out_ref[...] = acc_ref[...] + bd_ref[...]


def kernel(x, W_enc_parent, b_enc_parent, W_enc_leaf, b_enc_leaf, W_dec_leaf, b_dec):
    B, D_in = x.shape
    N_par = W_enc_parent.shape[1]
    D_leaf = W_enc_leaf.shape[1]
    be_p = b_enc_parent.reshape(1, N_par)
    be_l = b_enc_leaf.reshape(1, D_leaf)
    bd = b_dec.reshape(1, D_in)
    wp = W_enc_parent.astype(jnp.bfloat16)
    wl = W_enc_leaf.astype(jnp.bfloat16)
    wd = W_dec_leaf.astype(jnp.bfloat16)

    bbp = min(512, B)
    z_parent = pl.pallas_call(
        _parent_body,
        grid=(B // bbp,),
        in_specs=[
            pl.BlockSpec((bbp, D_in), lambda i: (i, 0)),
            pl.BlockSpec((D_in, N_par), lambda i: (0, 0)),
            pl.BlockSpec((1, N_par), lambda i: (0, 0)),
            pl.BlockSpec((1, D_in), lambda i: (0, 0)),
        ],
        out_specs=pl.BlockSpec((bbp, N_par), lambda i: (i, 0)),
        out_shape=jax.ShapeDtypeStruct((B, N_par), jnp.float32),
        compiler_params=pltpu.CompilerParams(
            dimension_semantics=("parallel",)),
    )(x, wp, be_p, bd)

    bb = min(1024, B)
    cb = min(1024, D_leaf)
    nj = D_leaf // cb
    ncm = nj * 128
    pre_leaf, cm = pl.pallas_call(
        _leaf_enc_body,
        grid=(B // bb, nj),
        in_specs=[
            pl.BlockSpec((bb, D_in), lambda i, j: (i, 0)),
            pl.BlockSpec((D_in, cb), lambda i, j: (0, j)),
            pl.BlockSpec((1, cb), lambda i, j: (0, j)),
            pl.BlockSpec((1, D_in), lambda i, j: (0, 0)),
        ],
        out_specs=[
            pl.BlockSpec((bb, cb), lambda i, j: (i, j)),
            pl.BlockSpec((bb, 128), lambda i, j: (i, j)),
        ],
        out_shape=[
            jax.ShapeDtypeStruct((B, D_leaf), jnp.float32),
            jax.ShapeDtypeStruct((B, ncm), jnp.float32),
        ],
        compiler_params=pltpu.CompilerParams(
            dimension_semantics=("parallel", "arbitrary")),
    )(x, wl, be_l, bd)

    bt = min(128, B)
    z_leaf, z_bf = pl.pallas_call(
        functools.partial(_topk_body, kk=_TOPK),
        grid=(B // bt,),
        in_specs=[
            pl.BlockSpec((bt, D_leaf), lambda i: (i, 0)),
            pl.BlockSpec((bt, ncm), lambda i: (i, 0)),
        ],
        out_specs=[
            pl.BlockSpec((bt, D_leaf), lambda i: (i, 0)),
            pl.BlockSpec((bt, D_leaf), lambda i: (i, 0)),
        ],
        out_shape=[
            jax.ShapeDtypeStruct((B, D_leaf), jnp.float32),
            jax.ShapeDtypeStruct((B, D_leaf), jnp.bfloat16),
        ],
        compiler_params=pltpu.CompilerParams(
            dimension_semantics=("parallel",)),
    )(pre_leaf, cm)

    bb2 = min(1024, B)
    ck = min(2048, D_leaf)
    nk = D_leaf // ck
    x_hat = pl.pallas_call(
        functools.partial(_decode_body, nk=nk),
        grid=(B // bb2, nk),
        in_specs=[
            pl.BlockSpec((bb2, ck), lambda i, k: (i, k)),
            pl.BlockSpec((ck, D_in), lambda i, k: (k, 0)),
            pl.BlockSpec((1, D_in), lambda i, k: (0, 0)),
        ],
        out_specs=pl.BlockSpec((bb2, D_in), lambda i, k: (i, 0)),
        out_shape=jax.ShapeDtypeStruct((B, D_in), jnp.float32),
        scratch_shapes=[pltpu.VMEM((bb2, D_in), jnp.float32)],
        compiler_params=pltpu.CompilerParams(
            dimension_semantics=("parallel", "arbitrary")),
    )(z_bf, wd, bd)

    return (x_hat, z_parent, z_leaf)


# speedup vs baseline: 9.7075x; 1.3039x over previous
import functools

import jax
import jax.numpy as jnp
from jax.experimental import pallas as pl
from jax.experimental.pallas import tpu as pltpu

_TOPK = 32


def _dot(a, b):
    return jax.lax.dot_general(a, b, (((1,), (0,)), ((), ())),
                               preferred_element_type=jnp.float32)


def _parent_body(x_ref, w_ref, be_ref, bd_ref, out_ref):
    xc = (x_ref[...] - bd_ref[...]).astype(jnp.bfloat16)
    pre = jnp.maximum(_dot(xc, w_ref[...]) + be_ref[...], 0.0)
    rowmax = jnp.max(pre, axis=1, keepdims=True)
    ids = jax.lax.broadcasted_iota(jnp.int32, pre.shape, 1)
    cand = jnp.where(pre == rowmax, ids, jnp.int32(pre.shape[1]))
    amin = jnp.min(cand, axis=1, keepdims=True)
    out_ref[...] = jnp.where(ids == amin, rowmax, 0.0)


def _leaf_enc_body(x_ref, w_ref, be_ref, bd_ref, pre_ref, cm_ref):
    xc = (x_ref[...] - bd_ref[...]).astype(jnp.bfloat16)
    pre = jnp.maximum(_dot(xc, w_ref[...]) + be_ref[...], 0.0)
    pre_ref[...] = pre
    cb = pre.shape[1]
    m1 = pre[:, 0:128]
    m2 = jnp.zeros_like(m1)
    for s in range(1, cb // 128):
        blk = pre[:, s * 128:(s + 1) * 128]
        m2 = jnp.maximum(m2, jnp.minimum(m1, blk))
        m1 = jnp.maximum(m1, blk)
    cm_ref[:, 0:128] = m1
    cm_ref[:, 128:256] = m2


def _topk_body(pre_ref, cm_ref, out_ref, obf_ref, *, kk):
    full = pre_ref[...]
    xi = jax.lax.bitcast_convert_type(full, jnp.int32)
    cmi = jax.lax.bitcast_convert_type(cm_ref[...], jnp.int32)
    hi0 = jnp.max(cmi, axis=1, keepdims=True) + 1
    lo0 = jnp.zeros_like(hi0)

    def cond1(carry):
        it, _, _, act = carry
        return jnp.logical_and(it < 31, act)

    def body1(carry):
        it, lo, hi, _ = carry
        mid = lo + (hi - lo) // 2
        cnt = jnp.sum((cmi >= mid).astype(jnp.int32), axis=1, keepdims=True)
        ge = cnt >= kk
        eq = cnt == kk
        nlo = jnp.where(ge, mid, lo)
        nhi = jnp.where(eq, mid + 1, jnp.where(ge, hi, mid))
        act = jnp.any((nhi - nlo) > 1)
        return it + 1, nlo, nhi, act

    act1 = jnp.any((hi0 - lo0) > 1)
    _, cand, _, _ = jax.lax.while_loop(
        cond1, body1, (jnp.int32(0), lo0, hi0, act1))

    pos = jnp.sum((xi >= 1).astype(jnp.int32), axis=1, keepdims=True)
    trivial = pos <= kk
    cand = jnp.where(trivial, 0, cand)

    cnt0 = jnp.sum((xi >= cand).astype(jnp.int32), axis=1, keepdims=True)
    ok = jnp.logical_or(trivial, cnt0 == kk)

    lo2 = cand
    hi2 = jnp.where(ok, cand + 1, hi0)

    def body2(carry):
        it, lo, hi, _ = carry
        mid = lo + (hi - lo) // 2
        cnt = jnp.sum((xi >= mid).astype(jnp.int32), axis=1, keepdims=True)
        ge = cnt >= kk
        eq = cnt == kk
        nlo = jnp.where(ge, mid, lo)
        nhi = jnp.where(eq, mid + 1, jnp.where(ge, hi, mid))
        act = jnp.any((nhi - nlo) > 1)
        return it + 1, nlo, nhi, act

    act2 = jnp.any((hi2 - lo2) > 1)
    _, lof, _, _ = jax.lax.while_loop(
        cond1, body2, (jnp.int32(0), lo2, hi2, act2))
    lof = jnp.where(ok, cand, lof)
    z = jnp.where(xi >= lof, full, 0.0)
    out_ref[...] = z
    obf_ref[...] = z.astype(jnp.bfloat16)


def _decode_body(z_ref, w_ref, bd_ref, out_ref, acc_ref, *, nk):
    k = pl.program_id(1)

    @pl.when(k == 0)
    def _():
        acc_ref[...] = jnp.zeros_like(acc_ref)

    acc_ref[...] += _dot(z_ref[...], w_ref[...])

    @pl.when(k == nk - 1)
    def _():
        out_ref[...] = acc_ref[...] + bd_ref[...]


def kernel(x, W_enc_parent, b_enc_parent, W_enc_leaf, b_enc_leaf, W_dec_leaf, b_dec):
    B, D_in = x.shape
    N_par = W_enc_parent.shape[1]
    D_leaf = W_enc_leaf.shape[1]
    be_p = b_enc_parent.reshape(1, N_par)
    be_l = b_enc_leaf.reshape(1, D_leaf)
    bd = b_dec.reshape(1, D_in)
    wp = W_enc_parent.astype(jnp.bfloat16)
    wl = W_enc_leaf.astype(jnp.bfloat16)
    wd = W_dec_leaf.astype(jnp.bfloat16)

    bbp = min(512, B)
    z_parent = pl.pallas_call(
        _parent_body,
        grid=(B // bbp,),
        in_specs=[
            pl.BlockSpec((bbp, D_in), lambda i: (i, 0)),
            pl.BlockSpec((D_in, N_par), lambda i: (0, 0)),
            pl.BlockSpec((1, N_par), lambda i: (0, 0)),
            pl.BlockSpec((1, D_in), lambda i: (0, 0)),
        ],
        out_specs=pl.BlockSpec((bbp, N_par), lambda i: (i, 0)),
        out_shape=jax.ShapeDtypeStruct((B, N_par), jnp.float32),
        compiler_params=pltpu.CompilerParams(
            dimension_semantics=("parallel",)),
    )(x, wp, be_p, bd)

    bb = min(1024, B)
    cb = min(1024, D_leaf)
    nj = D_leaf // cb
    ncm = nj * 256
    pre_leaf, cm = pl.pallas_call(
        _leaf_enc_body,
        grid=(B // bb, nj),
        in_specs=[
            pl.BlockSpec((bb, D_in), lambda i, j: (i, 0)),
            pl.BlockSpec((D_in, cb), lambda i, j: (0, j)),
            pl.BlockSpec((1, cb), lambda i, j: (0, j)),
            pl.BlockSpec((1, D_in), lambda i, j: (0, 0)),
        ],
        out_specs=[
            pl.BlockSpec((bb, cb), lambda i, j: (i, j)),
            pl.BlockSpec((bb, 256), lambda i, j: (i, j)),
        ],
        out_shape=[
            jax.ShapeDtypeStruct((B, D_leaf), jnp.float32),
            jax.ShapeDtypeStruct((B, ncm), jnp.float32),
        ],
        compiler_params=pltpu.CompilerParams(
            dimension_semantics=("parallel", "arbitrary")),
    )(x, wl, be_l, bd)

    bt = min(128, B)
    z_leaf, z_bf = pl.pallas_call(
        functools.partial(_topk_body, kk=_TOPK),
        grid=(B // bt,),
        in_specs=[
            pl.BlockSpec((bt, D_leaf), lambda i: (i, 0)),
            pl.BlockSpec((bt, ncm), lambda i: (i, 0)),
        ],
        out_specs=[
            pl.BlockSpec((bt, D_leaf), lambda i: (i, 0)),
            pl.BlockSpec((bt, D_leaf), lambda i: (i, 0)),
        ],
        out_shape=[
            jax.ShapeDtypeStruct((B, D_leaf), jnp.float32),
            jax.ShapeDtypeStruct((B, D_leaf), jnp.bfloat16),
        ],
        compiler_params=pltpu.CompilerParams(
            dimension_semantics=("parallel",)),
    )(pre_leaf, cm)

    bb2 = min(1024, B)
    ck = min(2048, D_leaf)
    nk = D_leaf // ck
    x_hat = pl.pallas_call(
        functools.partial(_decode_body, nk=nk),
        grid=(B // bb2, nk),
        in_specs=[
            pl.BlockSpec((bb2, ck), lambda i, k: (i, k)),
            pl.BlockSpec((ck, D_in), lambda i, k: (k, 0)),
            pl.BlockSpec((1, D_in), lambda i, k: (0, 0)),
        ],
        out_specs=pl.BlockSpec((bb2, D_in), lambda i, k: (i, 0)),
        out_shape=jax.ShapeDtypeStruct((B, D_in), jnp.float32),
        scratch_shapes=[pltpu.VMEM((bb2, D_in), jnp.float32)],
        compiler_params=pltpu.CompilerParams(
            dimension_semantics=("parallel", "arbitrary")),
    )(z_bf, wd, bd)

    return (x_hat, z_parent, z_leaf)
